# baseline (speedup 1.0000x reference)
import functools
import math

import jax
import jax.numpy as jnp
from jax import lax
from jax.experimental import pallas as pl
from jax.experimental.pallas import tpu as pltpu
from jax.experimental.pallas import tpu_sc as plsc

D_MODEL = 64
CHUNK = 128
SCALE = math.sqrt(D_MODEL)


@functools.cache
def _build(n_flat: int):
    info = plsc.get_sparse_core_info()
    nc, ns = info.num_cores, info.num_subcores
    nw = nc * ns
    n_chunks = n_flat // CHUNK
    assert n_chunks % nw == 0
    cpt = n_chunks // nw
    orows_per_chunk = CHUNK * D_MODEL // 128

    mesh = plsc.VectorSubcoreMesh(core_axis_name="c", subcore_axis_name="s")

    @functools.partial(
        pl.kernel,
        mesh=mesh,
        out_type=jax.ShapeDtypeStruct((n_flat * D_MODEL // 128, 128), jnp.float32),
        scratch_types=[
            pltpu.VMEM((cpt, CHUNK), jnp.int32),
            pltpu.VMEM((CHUNK,), jnp.int32),
            pltpu.VMEM((CHUNK,), jnp.int32),
            pltpu.VMEM((CHUNK, 128), jnp.float32),
            pltpu.VMEM((orows_per_chunk, 128), jnp.float32),
            pltpu.SemaphoreType.DMA,
        ],
    )
    def emb(x_hbm, table_hbm, out_hbm, idx_v, jrow, hoff, g, o, gsem):
        wid = lax.axis_index("s") * nc + lax.axis_index("c")
        base_chunk = wid * cpt
        pltpu.sync_copy(x_hbm.at[pl.ds(base_chunk, cpt), :], idx_v)

        def chunk_body(j, carry):
            for t in range(CHUNK // 16):
                v = idx_v[j, pl.ds(t * 16, 16)]
                jrow[pl.ds(t * 16, 16)] = lax.shift_right_logical(v, 1)
                hoff[pl.ds(t * 16, 16)] = (v & 1) * D_MODEL
            pltpu.async_copy(table_hbm.at[jrow], g, gsem).wait()

            def grp_body(t, c2):
                hv = hoff[pl.ds(t * 16, 16)]
                for l in range(16):
                    off = hv[l]
                    r = t * 16 + l
                    orow = t * 8 + l // 2
                    opar = (l % 2) * D_MODEL
                    for q in range(D_MODEL // 16):
                        o[orow, pl.ds(opar + q * 16, 16)] = (
                            g[r, pl.ds(off + q * 16, 16)] * SCALE
                        )
                return c2

            lax.fori_loop(0, CHUNK // 16, grp_body, 0)
            pltpu.sync_copy(
                o,
                out_hbm.at[pl.ds((base_chunk + j) * orows_per_chunk,
                                 orows_per_chunk), :],
            )
            return carry

        lax.fori_loop(0, cpt, chunk_body, 0)

    return emb


def kernel(x, table):
    b, h = x.shape
    n_flat = b * h
    v, d = table.shape
    x2 = x.reshape(n_flat // CHUNK, CHUNK)
    table2 = table.reshape(v // 2, 128)
    out = _build(n_flat)(x2, table2)
    return out.reshape(b, h, D_MODEL)


# speedup vs baseline: 1.3608x; 1.3608x over previous
import functools
import math

import jax
import jax.numpy as jnp
from jax import lax
from jax.experimental import pallas as pl
from jax.experimental.pallas import tpu as pltpu
from jax.experimental.pallas import tpu_sc as plsc

D_MODEL = 64
CHUNK = 128
SCALE = math.sqrt(D_MODEL)


@functools.cache
def _build(n_flat: int):
    info = plsc.get_sparse_core_info()
    nc, ns = info.num_cores, info.num_subcores
    nw = nc * ns
    n_chunks = n_flat // CHUNK
    assert n_chunks % nw == 0
    cpt = n_chunks // nw

    mesh = plsc.VectorSubcoreMesh(core_axis_name="c", subcore_axis_name="s")

    @functools.partial(
        pl.kernel,
        mesh=mesh,
        out_type=jax.ShapeDtypeStruct((n_flat, D_MODEL), jnp.float32),
        compiler_params=pltpu.CompilerParams(use_tc_tiling_on_sc=False),
        scratch_types=[
            pltpu.VMEM((cpt, CHUNK), jnp.int32),
            pltpu.VMEM((CHUNK, D_MODEL), jnp.float32),
            pltpu.SemaphoreType.DMA,
        ],
    )
    def emb(x_hbm, table_hbm, out_hbm, idx_v, rows_v, gsem):
        wid = lax.axis_index("s") * nc + lax.axis_index("c")
        base_chunk = wid * cpt
        pltpu.sync_copy(x_hbm.at[pl.ds(base_chunk, cpt), :], idx_v)

        def chunk_body(j, carry):
            pltpu.async_copy(table_hbm.at[idx_v.at[j]], rows_v, gsem).wait()

            def row_body(r, c2):
                for q in range(D_MODEL // 16):
                    rows_v[r, pl.ds(q * 16, 16)] = (
                        rows_v[r, pl.ds(q * 16, 16)] * SCALE
                    )
                return c2

            lax.fori_loop(0, CHUNK, row_body, 0, unroll=2)
            pltpu.sync_copy(
                rows_v, out_hbm.at[pl.ds((base_chunk + j) * CHUNK, CHUNK), :]
            )
            return carry

        lax.fori_loop(0, cpt, chunk_body, 0)

    return emb


def kernel(x, table):
    b, h = x.shape
    n_flat = b * h
    x2 = x.reshape(n_flat // CHUNK, CHUNK)
    out = _build(n_flat)(x2, table)
    return out.reshape(b, h, D_MODEL)


# speedup vs baseline: 1.5763x; 1.1583x over previous
import functools
import math

import jax
import jax.numpy as jnp
from jax import lax
from jax.experimental import pallas as pl
from jax.experimental.pallas import tpu as pltpu
from jax.experimental.pallas import tpu_sc as plsc

D_MODEL = 64
SUB = 40
NSUB = 5
NBUF = 10
LOOK = 5
SCALE = math.sqrt(D_MODEL)


@functools.cache
def _build(batch: int, hist: int):
    info = plsc.get_sparse_core_info()
    nc, ns = info.num_cores, info.num_subcores
    nw = nc * ns
    assert batch % nw == 0 and hist == NSUB * SUB
    bpt = batch // nw
    steps = bpt * NSUB
    assert steps % NBUF == 0

    mesh = plsc.VectorSubcoreMesh(core_axis_name="c", subcore_axis_name="s")

    @functools.partial(
        pl.kernel,
        mesh=mesh,
        out_type=jax.ShapeDtypeStruct((batch, hist, D_MODEL), jnp.float32),
        compiler_params=pltpu.CompilerParams(use_tc_tiling_on_sc=False),
        scratch_types=[
            pltpu.VMEM((bpt, hist), jnp.int32),
            [pltpu.VMEM((SUB, D_MODEL), jnp.float32) for _ in range(NBUF)],
            [pltpu.SemaphoreType.DMA for _ in range(NBUF)],
            [pltpu.SemaphoreType.DMA for _ in range(NBUF)],
        ],
    )
    def emb(x_hbm, table_hbm, out_hbm, idx_v, gbufs, gsems, ssems):
        wid = lax.axis_index("s") * nc + lax.axis_index("c")
        b0 = wid * bpt
        pltpu.sync_copy(x_hbm.at[pl.ds(b0, bpt), :], idx_v)

        def gather(step, row_off, col, slot):
            pltpu.async_copy(
                table_hbm.at[idx_v.at[row_off, pl.ds(col * SUB, SUB)]],
                gbufs[slot],
                gsems[slot],
            )

        for s in range(LOOK):
            gather(s, s // NSUB, s % NSUB, s % NBUF)

        def outer(r2, carry):
            for u in range(NBUF):
                s = r2 * NBUF + u
                row = 2 * r2 + u // NSUB
                col = u % NSUB
                slot = u

                nslot = (u + LOOK) % NBUF

                @pl.when(s + LOOK < steps)
                def _():
                    @pl.when(s >= LOOK)
                    def _():
                        pltpu.make_async_copy(
                            gbufs[nslot],
                            out_hbm.at[0, pl.ds(0, SUB), :],
                            ssems[nslot],
                        ).wait()

                    sn = s + LOOK
                    gather(sn, 2 * r2 + (u + LOOK) // NSUB,
                           (u + LOOK) % NSUB, nslot)

                pltpu.make_async_copy(
                    table_hbm.at[idx_v.at[row, pl.ds(col * SUB, SUB)]],
                    gbufs[slot],
                    gsems[slot],
                ).wait()

                def row_body(r, c2):
                    for q in range(D_MODEL // 16):
                        gbufs[slot][r, pl.ds(q * 16, 16)] = (
                            gbufs[slot][r, pl.ds(q * 16, 16)] * SCALE
                        )
                    return c2

                lax.fori_loop(0, SUB, row_body, 0, unroll=2)

                pltpu.async_copy(
                    gbufs[slot],
                    out_hbm.at[b0 + row, pl.ds(col * SUB, SUB), :],
                    ssems[slot],
                )
            return carry

        lax.fori_loop(0, steps // NBUF, outer, 0)

        for u in range(NBUF):
            pltpu.make_async_copy(
                gbufs[u],
                out_hbm.at[0, pl.ds(0, SUB), :],
                ssems[u],
            ).wait()

    return emb


def kernel(x, table):
    b, h = x.shape
    return _build(b, h)(x, table)
